# baseline (speedup 1.0000x reference)
import jax
import jax.numpy as jnp
from jax.experimental import pallas as pl
from jax.experimental.pallas import tpu as pltpu

B, D, C = 32, 8732, 21
ALPHA = 1.0
NEG_FACTOR = 3.0
N_ITERS = 28


def _ssd_step(pos_ref, x_ref, gloc_ref, gconf_ref, out_ref,
              pn_s, l1p_s, pos_s):
    b = pl.program_id(0)

    pos = pos_ref[0]
    xt = x_ref[0].T
    xloc = xt[:4]
    xconf = xt[4:]
    gloc = gloc_ref[0].T
    gconf = gconf_ref[0].T

    d = xloc - gloc
    ad = jnp.abs(d)
    sl1 = jnp.where(ad < 1.0, 0.5 * d * d, ad - 0.5)
    l1 = jnp.sum(sl1, axis=0, keepdims=True)

    e = jnp.exp(xconf)
    lse = jnp.log(jnp.sum(e, axis=0, keepdims=True))
    gsum = jnp.sum(gconf, axis=0, keepdims=True)
    dot = jnp.sum(gconf * xconf, axis=0, keepdims=True)
    pn = gsum * lse - dot

    pn_s[pl.ds(b, 1), :] = pn
    l1p_s[pl.ds(b, 1), :] = l1 * pos
    pos_s[pl.ds(b, 1), :] = pos

    @pl.when(b == B - 1)
    def _final():
        pn_all = pn_s[:, :]
        posa = pos_s[:, :]
        l1pa = l1p_s[:, :]

        n_pos = jnp.sum(posa, axis=1, keepdims=True)
        p_sum = jnp.sum(pn_all * posa, axis=1, keepdims=True)
        l1_sum = jnp.sum(l1pa, axis=1, keepdims=True)
        neg_cnt = jnp.float32(D) - n_pos
        k = jnp.minimum(neg_cnt, NEG_FACTOR * n_pos)

        negv = jnp.where(posa > 0.5, -1.0, pn_all)

        lo0 = jnp.full((B, 1), -0.5, jnp.float32)
        hi0 = jnp.max(negv, axis=1, keepdims=True) + 1.0

        def body(_, carry):
            lo, hi = carry
            mid = 0.5 * (lo + hi)
            cnt = jnp.sum(jnp.where(negv > mid, 1.0, 0.0), axis=1,
                          keepdims=True)
            ge = cnt >= k
            return jnp.where(ge, mid, lo), jnp.where(ge, hi, mid)

        lo, hi = jax.lax.fori_loop(0, N_ITERS, body, (lo0, hi0))
        gt = negv > hi
        c = jnp.sum(jnp.where(gt, 1.0, 0.0), axis=1, keepdims=True)
        sum_gt = jnp.sum(jnp.where(gt, negv, 0.0), axis=1, keepdims=True)
        n_sum = sum_gt + (k - c) * hi
        n_sum = jnp.where(k > 0.0, n_sum, 0.0)

        safe_n = jnp.maximum(n_pos, 1.0)
        has_pos = n_pos > 0.0
        conf_loss = jnp.where(has_pos, (p_sum + n_sum) / safe_n, 0.0)
        loc_loss = jnp.where(has_pos, l1_sum / safe_n, 0.0)
        total = jnp.sum(conf_loss + ALPHA * loc_loss, axis=0,
                        keepdims=True) / jnp.float32(B)
        out_ref[:, :] = total


def kernel(pos_indicator, predicts, gt_loc, gt_conf):
    posf = pos_indicator.astype(jnp.float32)[:, None, :]

    out = pl.pallas_call(
        _ssd_step,
        grid=(B,),
        in_specs=[
            pl.BlockSpec((1, 1, D), lambda b: (b, 0, 0)),
            pl.BlockSpec((1, D, 4 + C), lambda b: (b, 0, 0)),
            pl.BlockSpec((1, D, 4), lambda b: (b, 0, 0)),
            pl.BlockSpec((1, D, C), lambda b: (b, 0, 0)),
        ],
        out_specs=pl.BlockSpec((1, 1), lambda b: (0, 0)),
        out_shape=jax.ShapeDtypeStruct((1, 1), jnp.float32),
        scratch_shapes=[
            pltpu.VMEM((B, D), jnp.float32),
            pltpu.VMEM((B, D), jnp.float32),
            pltpu.VMEM((B, D), jnp.float32),
        ],
    )(posf, predicts, gt_loc, gt_conf)
    return out[0, 0]


# speedup vs baseline: 2.9975x; 2.9975x over previous
import jax
import jax.numpy as jnp
from jax.experimental import pallas as pl
from jax.experimental.pallas import tpu as pltpu

B, D, C = 32, 8732, 21
ALPHA = 1.0
NEG_FACTOR = 3.0
N_ITERS = 26


def _ssd_step(pos_ref, xall_ref, gloc_ref, gconf_ref, out_ref,
              neg_s, stats_s):
    b = pl.program_id(0)

    pos = pos_ref[0]
    xall = xall_ref[0]
    xloc = xall[:4]
    xconf = xall[4:]
    gloc = gloc_ref[0]
    gconf = gconf_ref[0]

    d = xloc - gloc
    ad = jnp.abs(d)
    sl1 = jnp.where(ad < 1.0, 0.5 * d * d, ad - 0.5)
    l1 = jnp.sum(sl1, axis=0, keepdims=True)

    e = jnp.exp(xconf)
    lse = jnp.log(jnp.sum(e, axis=0, keepdims=True))
    gsum = jnp.sum(gconf, axis=0, keepdims=True)
    dot = jnp.sum(gconf * xconf, axis=0, keepdims=True)
    pn = gsum * lse - dot

    is_pos = pos > 0.5
    n_pos = jnp.sum(pos, axis=1, keepdims=True)
    p_sum = jnp.sum(jnp.where(is_pos, pn, 0.0), axis=1, keepdims=True)
    l1_sum = jnp.sum(jnp.where(is_pos, l1, 0.0), axis=1, keepdims=True)

    neg_s[pl.ds(b, 1), :] = jnp.where(is_pos, -1.0, pn)

    lane = jax.lax.broadcasted_iota(jnp.int32, (1, 128), 1)
    stats = jnp.where(lane == 0, n_pos,
                      jnp.where(lane == 1, p_sum,
                                jnp.where(lane == 2, l1_sum, 0.0)))
    stats_s[pl.ds(b, 1), :] = stats

    @pl.when(b == B - 1)
    def _final():
        negv = neg_s[:, :]
        stats = stats_s[:, :]
        n_pos = stats[:, 0:1]
        p_sum = stats[:, 1:2]
        l1_sum = stats[:, 2:3]
        neg_cnt = jnp.float32(D) - n_pos
        k = jnp.minimum(neg_cnt, NEG_FACTOR * n_pos)

        lo0 = jnp.full((B, 1), -0.5, jnp.float32)
        hi0 = jnp.max(negv, axis=1, keepdims=True) + 1.0

        def body(_, carry):
            lo, hi = carry
            mid = 0.5 * (lo + hi)
            cnt = jnp.sum(jnp.where(negv > mid, 1.0, 0.0), axis=1,
                          keepdims=True)
            ge = cnt >= k
            return jnp.where(ge, mid, lo), jnp.where(ge, hi, mid)

        lo, hi = jax.lax.fori_loop(0, N_ITERS, body, (lo0, hi0))
        gt = negv > hi
        c = jnp.sum(jnp.where(gt, 1.0, 0.0), axis=1, keepdims=True)
        sum_gt = jnp.sum(jnp.where(gt, negv, 0.0), axis=1, keepdims=True)
        n_sum = sum_gt + (k - c) * hi
        n_sum = jnp.where(k > 0.0, n_sum, 0.0)

        safe_n = jnp.maximum(n_pos, 1.0)
        has_pos = n_pos > 0.0
        conf_loss = jnp.where(has_pos, (p_sum + n_sum) / safe_n, 0.0)
        loc_loss = jnp.where(has_pos, l1_sum / safe_n, 0.0)
        total = jnp.sum(conf_loss + ALPHA * loc_loss, axis=0,
                        keepdims=True) / jnp.float32(B)
        out_ref[:, :] = total


def kernel(pos_indicator, predicts, gt_loc, gt_conf):
    posf = pos_indicator.astype(jnp.float32)[:, None, :]
    xall_t = jnp.transpose(predicts, (0, 2, 1))
    gloc_t = jnp.transpose(gt_loc, (0, 2, 1))
    gconf_t = jnp.transpose(gt_conf, (0, 2, 1))

    out = pl.pallas_call(
        _ssd_step,
        grid=(B,),
        in_specs=[
            pl.BlockSpec((1, 1, D), lambda b: (b, 0, 0)),
            pl.BlockSpec((1, 4 + C, D), lambda b: (b, 0, 0)),
            pl.BlockSpec((1, 4, D), lambda b: (b, 0, 0)),
            pl.BlockSpec((1, C, D), lambda b: (b, 0, 0)),
        ],
        out_specs=pl.BlockSpec((1, 1), lambda b: (0, 0)),
        out_shape=jax.ShapeDtypeStruct((1, 1), jnp.float32),
        scratch_shapes=[
            pltpu.VMEM((B, D), jnp.float32),
            pltpu.VMEM((B, 128), jnp.float32),
        ],
    )(posf, xall_t, gloc_t, gconf_t)
    return out[0, 0]


# speedup vs baseline: 3.6668x; 1.2233x over previous
import jax
import jax.numpy as jnp
from jax.experimental import pallas as pl
from jax.experimental.pallas import tpu as pltpu

B, D, C = 32, 8732, 21
ALPHA = 1.0
NEG_FACTOR = 3.0
N_ITERS = 26


def _ssd_step(pos_ref, xloc_ref, xconf_ref, gloc_ref, gconf_ref, out_ref,
              neg_s, stats_s):
    b = pl.program_id(0)

    pos = pos_ref[0]
    xloc = xloc_ref[0]
    xconf = xconf_ref[0]
    gloc = gloc_ref[0]
    gconf = gconf_ref[0]

    d = xloc - gloc
    ad = jnp.abs(d)
    sl1 = jnp.where(ad < 1.0, 0.5 * d * d, ad - 0.5)
    l1 = jnp.sum(sl1, axis=0, keepdims=True)

    e = jnp.exp(xconf)
    lse = jnp.log(jnp.sum(e, axis=0, keepdims=True))
    gsum = jnp.sum(gconf, axis=0, keepdims=True)
    dot = jnp.sum(gconf * xconf, axis=0, keepdims=True)
    pn = gsum * lse - dot

    is_pos = pos > 0.5
    n_pos = jnp.sum(pos, axis=1, keepdims=True)
    p_sum = jnp.sum(jnp.where(is_pos, pn, 0.0), axis=1, keepdims=True)
    l1_sum = jnp.sum(jnp.where(is_pos, l1, 0.0), axis=1, keepdims=True)

    neg_s[pl.ds(b, 1), :] = jnp.where(is_pos, -1.0, pn)

    lane = jax.lax.broadcasted_iota(jnp.int32, (1, 128), 1)
    stats = jnp.where(lane == 0, n_pos,
                      jnp.where(lane == 1, p_sum,
                                jnp.where(lane == 2, l1_sum, 0.0)))
    stats_s[pl.ds(b, 1), :] = stats

    @pl.when(b == B - 1)
    def _final():
        negv = neg_s[:, :]
        stats = stats_s[:, :]
        n_pos = stats[:, 0:1]
        p_sum = stats[:, 1:2]
        l1_sum = stats[:, 2:3]
        neg_cnt = jnp.float32(D) - n_pos
        k = jnp.minimum(neg_cnt, NEG_FACTOR * n_pos)

        lo0 = jnp.full((B, 1), -0.5, jnp.float32)
        hi0 = jnp.max(negv, axis=1, keepdims=True) + 1.0

        def body(_, carry):
            lo, hi = carry
            mid = 0.5 * (lo + hi)
            cnt = jnp.sum(jnp.where(negv > mid, 1.0, 0.0), axis=1,
                          keepdims=True)
            ge = cnt >= k
            return jnp.where(ge, mid, lo), jnp.where(ge, hi, mid)

        lo, hi = jax.lax.fori_loop(0, N_ITERS, body, (lo0, hi0))
        gt = negv > hi
        c = jnp.sum(jnp.where(gt, 1.0, 0.0), axis=1, keepdims=True)
        sum_gt = jnp.sum(jnp.where(gt, negv, 0.0), axis=1, keepdims=True)
        n_sum = sum_gt + (k - c) * hi
        n_sum = jnp.where(k > 0.0, n_sum, 0.0)

        safe_n = jnp.maximum(n_pos, 1.0)
        has_pos = n_pos > 0.0
        conf_loss = jnp.where(has_pos, (p_sum + n_sum) / safe_n, 0.0)
        loc_loss = jnp.where(has_pos, l1_sum / safe_n, 0.0)
        total = jnp.sum(conf_loss + ALPHA * loc_loss, axis=0,
                        keepdims=True) / jnp.float32(B)
        out_ref[:, :] = total


def kernel(pos_indicator, predicts, gt_loc, gt_conf):
    posf = pos_indicator.astype(jnp.float32)[:, None, :]
    xloc_t = jnp.transpose(predicts[:, :, :4], (0, 2, 1))
    xconf_t = jnp.transpose(predicts[:, :, 4:], (0, 2, 1))
    gloc_t = jnp.transpose(gt_loc, (0, 2, 1))
    gconf_t = jnp.transpose(gt_conf, (0, 2, 1))

    out = pl.pallas_call(
        _ssd_step,
        grid=(B,),
        in_specs=[
            pl.BlockSpec((1, 1, D), lambda b: (b, 0, 0)),
            pl.BlockSpec((1, 4, D), lambda b: (b, 0, 0)),
            pl.BlockSpec((1, C, D), lambda b: (b, 0, 0)),
            pl.BlockSpec((1, 4, D), lambda b: (b, 0, 0)),
            pl.BlockSpec((1, C, D), lambda b: (b, 0, 0)),
        ],
        out_specs=pl.BlockSpec((1, 1), lambda b: (0, 0)),
        out_shape=jax.ShapeDtypeStruct((1, 1), jnp.float32),
        scratch_shapes=[
            pltpu.VMEM((B, D), jnp.float32),
            pltpu.VMEM((B, 128), jnp.float32),
        ],
    )(posf, xloc_t, xconf_t, gloc_t, gconf_t)
    return out[0, 0]


# speedup vs baseline: 4.0698x; 1.1099x over previous
import jax
import jax.numpy as jnp
from jax.experimental import pallas as pl
from jax.experimental.pallas import tpu as pltpu

B, D, C = 32, 8732, 21
ALPHA = 1.0
NEG_FACTOR = 3.0
N_ITERS = 28


def _ssd_step(pos_ref, xloc_ref, xconf_ref, gloc_ref, gconf_ref, out_ref,
              pn_s, l1p_s, pos_s):
    b = pl.program_id(0)

    pos = pos_ref[0]
    xloc = xloc_ref[0]
    xconf = xconf_ref[0]
    gloc = gloc_ref[0]
    gconf = gconf_ref[0]

    d = xloc - gloc
    ad = jnp.abs(d)
    sl1 = jnp.where(ad < 1.0, jnp.bfloat16(0.5) * d * d,
                    ad - jnp.bfloat16(0.5))
    l1 = jnp.sum(sl1, axis=0, keepdims=True,
                 dtype=jnp.float32)

    e = jnp.exp(xconf)
    lse = jnp.log(jnp.sum(e, axis=0, keepdims=True,
                          dtype=jnp.float32))
    gsum = jnp.sum(gconf, axis=0, keepdims=True,
                   dtype=jnp.float32)
    dot = jnp.sum(gconf * xconf, axis=0, keepdims=True,
                  dtype=jnp.float32)
    pn = gsum * lse - dot

    pn_s[pl.ds(b, 1), :] = pn
    l1p_s[pl.ds(b, 1), :] = l1 * pos
    pos_s[pl.ds(b, 1), :] = pos

    @pl.when(b == B - 1)
    def _final():
        pn_all = pn_s[:, :]
        posa = pos_s[:, :]
        l1pa = l1p_s[:, :]

        n_pos = jnp.sum(posa, axis=1, keepdims=True)
        p_sum = jnp.sum(pn_all * posa, axis=1, keepdims=True)
        l1_sum = jnp.sum(l1pa, axis=1, keepdims=True)
        neg_cnt = jnp.float32(D) - n_pos
        k = jnp.minimum(neg_cnt, NEG_FACTOR * n_pos)

        negv = jnp.where(posa > 0.5, -1.0, pn_all)

        lo0 = jnp.full((B, 1), -0.5, jnp.float32)
        hi0 = jnp.max(negv, axis=1, keepdims=True) + 1.0

        def body(_, carry):
            lo, hi = carry
            mid = 0.5 * (lo + hi)
            cnt = jnp.sum(jnp.where(negv > mid, 1.0, 0.0), axis=1,
                          keepdims=True)
            ge = cnt >= k
            return jnp.where(ge, mid, lo), jnp.where(ge, hi, mid)

        lo, hi = jax.lax.fori_loop(0, N_ITERS, body, (lo0, hi0))
        gt = negv > hi
        c = jnp.sum(jnp.where(gt, 1.0, 0.0), axis=1, keepdims=True)
        sum_gt = jnp.sum(jnp.where(gt, negv, 0.0), axis=1, keepdims=True)
        n_sum = sum_gt + (k - c) * hi
        n_sum = jnp.where(k > 0.0, n_sum, 0.0)

        safe_n = jnp.maximum(n_pos, 1.0)
        has_pos = n_pos > 0.0
        conf_loss = jnp.where(has_pos, (p_sum + n_sum) / safe_n, 0.0)
        loc_loss = jnp.where(has_pos, l1_sum / safe_n, 0.0)
        total = jnp.sum(conf_loss + ALPHA * loc_loss, axis=0,
                        keepdims=True) / jnp.float32(B)
        out_ref[:, :] = total


def kernel(pos_indicator, predicts, gt_loc, gt_conf):
    bf = jnp.bfloat16
    posf = pos_indicator.astype(jnp.float32)[:, None, :]
    xloc = jnp.transpose(predicts[:, :, :4].astype(bf), (0, 2, 1))
    xconf = jnp.transpose(predicts[:, :, 4:].astype(bf), (0, 2, 1))
    gloc_t = jnp.transpose(gt_loc.astype(bf), (0, 2, 1))
    gconf_t = jnp.transpose(gt_conf.astype(bf), (0, 2, 1))

    out = pl.pallas_call(
        _ssd_step,
        grid=(B,),
        in_specs=[
            pl.BlockSpec((1, 1, D), lambda b: (b, 0, 0)),
            pl.BlockSpec((1, 4, D), lambda b: (b, 0, 0)),
            pl.BlockSpec((1, C, D), lambda b: (b, 0, 0)),
            pl.BlockSpec((1, 4, D), lambda b: (b, 0, 0)),
            pl.BlockSpec((1, C, D), lambda b: (b, 0, 0)),
        ],
        out_specs=pl.BlockSpec((1, 1), lambda b: (0, 0)),
        out_shape=jax.ShapeDtypeStruct((1, 1), jnp.float32),
        scratch_shapes=[
            pltpu.VMEM((B, D), jnp.float32),
            pltpu.VMEM((B, D), jnp.float32),
            pltpu.VMEM((B, D), jnp.float32),
        ],
    )(posf, xloc, xconf, gloc_t, gconf_t)
    return out[0, 0]


# speedup vs baseline: 4.1926x; 1.0302x over previous
import jax
import jax.numpy as jnp
from jax.experimental import pallas as pl
from jax.experimental.pallas import tpu as pltpu

B, D, C = 32, 8732, 21
ALPHA = 1.0
NEG_FACTOR = 3.0
N_ITERS = 16


def _ssd_step(pos_ref, xloc_ref, xconf_ref, gloc_ref, gconf_ref, out_ref,
              pn_s, l1p_s, pos_s):
    b = pl.program_id(0)

    pos = pos_ref[0]
    xloc = xloc_ref[0]
    xconf = xconf_ref[0]
    gloc = gloc_ref[0]
    gconf = gconf_ref[0]

    d = xloc - gloc
    ad = jnp.abs(d)
    sl1 = jnp.where(ad < 1.0, jnp.bfloat16(0.5) * d * d,
                    ad - jnp.bfloat16(0.5))
    l1 = jnp.sum(sl1, axis=0, keepdims=True,
                 dtype=jnp.float32)

    e = jnp.exp(xconf)
    lse = jnp.log(jnp.sum(e, axis=0, keepdims=True,
                          dtype=jnp.float32))
    gsum = jnp.sum(gconf, axis=0, keepdims=True,
                   dtype=jnp.float32)
    dot = jnp.sum(gconf * xconf, axis=0, keepdims=True,
                  dtype=jnp.float32)
    pn = gsum * lse - dot

    pn_s[pl.ds(b, 1), :] = pn
    l1p_s[pl.ds(b, 1), :] = l1 * pos
    pos_s[pl.ds(b, 1), :] = pos

    @pl.when(b == B - 1)
    def _final():
        pn_all = pn_s[:, :]
        posa = pos_s[:, :]
        l1pa = l1p_s[:, :]

        n_pos = jnp.sum(posa, axis=1, keepdims=True)
        p_sum = jnp.sum(pn_all * posa, axis=1, keepdims=True)
        l1_sum = jnp.sum(l1pa, axis=1, keepdims=True)
        neg_cnt = jnp.float32(D) - n_pos
        k = jnp.minimum(neg_cnt, NEG_FACTOR * n_pos)

        negv = jnp.where(posa > 0.5, -1.0, pn_all)

        lo0 = jnp.full((B, 1), -0.5, jnp.float32)
        hi0 = jnp.max(negv, axis=1, keepdims=True) + 1.0

        def body(_, carry):
            lo, hi = carry
            mid = 0.5 * (lo + hi)
            cnt = jnp.sum(jnp.where(negv > mid, 1.0, 0.0), axis=1,
                          keepdims=True)
            ge = cnt >= k
            return jnp.where(ge, mid, lo), jnp.where(ge, hi, mid)

        lo, hi = jax.lax.fori_loop(0, N_ITERS, body, (lo0, hi0))
        gt = negv > hi
        c = jnp.sum(jnp.where(gt, 1.0, 0.0), axis=1, keepdims=True)
        sum_gt = jnp.sum(jnp.where(gt, negv, 0.0), axis=1, keepdims=True)
        n_sum = sum_gt + (k - c) * hi
        n_sum = jnp.where(k > 0.0, n_sum, 0.0)

        safe_n = jnp.maximum(n_pos, 1.0)
        has_pos = n_pos > 0.0
        conf_loss = jnp.where(has_pos, (p_sum + n_sum) / safe_n, 0.0)
        loc_loss = jnp.where(has_pos, l1_sum / safe_n, 0.0)
        total = jnp.sum(conf_loss + ALPHA * loc_loss, axis=0,
                        keepdims=True) / jnp.float32(B)
        out_ref[:, :] = total


def kernel(pos_indicator, predicts, gt_loc, gt_conf):
    bf = jnp.bfloat16
    posf = pos_indicator.astype(jnp.float32)[:, None, :]
    xloc = jnp.transpose(predicts[:, :, :4].astype(bf), (0, 2, 1))
    xconf = jnp.transpose(predicts[:, :, 4:].astype(bf), (0, 2, 1))
    gloc_t = jnp.transpose(gt_loc.astype(bf), (0, 2, 1))
    gconf_t = jnp.transpose(gt_conf.astype(bf), (0, 2, 1))

    out = pl.pallas_call(
        _ssd_step,
        grid=(B,),
        in_specs=[
            pl.BlockSpec((1, 1, D), lambda b: (b, 0, 0)),
            pl.BlockSpec((1, 4, D), lambda b: (b, 0, 0)),
            pl.BlockSpec((1, C, D), lambda b: (b, 0, 0)),
            pl.BlockSpec((1, 4, D), lambda b: (b, 0, 0)),
            pl.BlockSpec((1, C, D), lambda b: (b, 0, 0)),
        ],
        out_specs=pl.BlockSpec((1, 1), lambda b: (0, 0)),
        out_shape=jax.ShapeDtypeStruct((1, 1), jnp.float32),
        scratch_shapes=[
            pltpu.VMEM((B, D), jnp.float32),
            pltpu.VMEM((B, D), jnp.float32),
            pltpu.VMEM((B, D), jnp.float32),
        ],
    )(posf, xloc, xconf, gloc_t, gconf_t)
    return out[0, 0]


# speedup vs baseline: 5.1518x; 1.2288x over previous
import jax
import jax.numpy as jnp
from jax.experimental import pallas as pl
from jax.experimental.pallas import tpu as pltpu

B, D, C = 32, 8732, 21
ALPHA = 1.0
NEG_FACTOR = 3.0
N_ITERS = 16


def _ssd_step(pos_ref, xloc_ref, xconf_ref, gloc_ref, gconf_ref, out_ref,
              pn_s, l1_s):
    b = pl.program_id(0)

    xloc = xloc_ref[0]
    xconf = xconf_ref[0]
    gloc = gloc_ref[0]
    gconf = gconf_ref[0]

    d = xloc - gloc
    ad = jnp.abs(d)
    sl1 = jnp.where(ad < 1.0, 0.5 * d * d, ad - 0.5)
    l1 = jnp.sum(sl1, axis=0, keepdims=True)

    e = jnp.exp(xconf)
    lse = jnp.log(jnp.sum(e, axis=0, keepdims=True,
                          dtype=jnp.float32))
    gsum = jnp.sum(gconf, axis=0, keepdims=True,
                   dtype=jnp.float32)
    dot = jnp.sum(gconf * xconf, axis=0, keepdims=True,
                  dtype=jnp.float32)
    pn = gsum * lse - dot

    pn_s[pl.ds(b, 1), :] = pn
    l1_s[pl.ds(b, 1), :] = l1

    @pl.when(b == B - 1)
    def _final():
        pn_all = pn_s[:, :]
        l1a = l1_s[:, :]
        posa = pos_ref[:, :]
        is_pos = posa > 0.5

        n_pos = jnp.sum(posa, axis=1, keepdims=True)
        p_sum = jnp.sum(jnp.where(is_pos, pn_all, 0.0), axis=1,
                        keepdims=True)
        l1_sum = jnp.sum(jnp.where(is_pos, l1a, 0.0), axis=1,
                         keepdims=True)
        neg_cnt = jnp.float32(D) - n_pos
        k = jnp.minimum(neg_cnt, NEG_FACTOR * n_pos)

        negv = jnp.where(is_pos, -1.0, pn_all)

        lo0 = jnp.full((B, 1), -0.5, jnp.float32)
        hi0 = jnp.max(negv, axis=1, keepdims=True) + 1.0

        def body(_, carry):
            lo, hi = carry
            mid = 0.5 * (lo + hi)
            cnt = jnp.sum(jnp.where(negv > mid, 1.0, 0.0), axis=1,
                          keepdims=True)
            ge = cnt >= k
            return jnp.where(ge, mid, lo), jnp.where(ge, hi, mid)

        lo, hi = jax.lax.fori_loop(0, N_ITERS, body, (lo0, hi0))
        gt = negv > hi
        c = jnp.sum(jnp.where(gt, 1.0, 0.0), axis=1, keepdims=True)
        sum_gt = jnp.sum(jnp.where(gt, negv, 0.0), axis=1, keepdims=True)
        n_sum = sum_gt + (k - c) * hi
        n_sum = jnp.where(k > 0.0, n_sum, 0.0)

        safe_n = jnp.maximum(n_pos, 1.0)
        has_pos = n_pos > 0.0
        conf_loss = jnp.where(has_pos, (p_sum + n_sum) / safe_n, 0.0)
        loc_loss = jnp.where(has_pos, l1_sum / safe_n, 0.0)
        total = jnp.sum(conf_loss + ALPHA * loc_loss, axis=0,
                        keepdims=True) / jnp.float32(B)
        out_ref[:, :] = total


def kernel(pos_indicator, predicts, gt_loc, gt_conf):
    bf = jnp.bfloat16
    posf = pos_indicator.astype(jnp.float32)
    xloc = jnp.transpose(predicts[:, :, :4], (0, 2, 1))
    xconf = jnp.transpose(predicts[:, :, 4:].astype(bf), (0, 2, 1))
    gloc_t = jnp.transpose(gt_loc, (0, 2, 1))
    gconf_t = jnp.transpose(gt_conf.astype(bf), (0, 2, 1))

    out = pl.pallas_call(
        _ssd_step,
        grid=(B,),
        in_specs=[
            pl.BlockSpec((B, D), lambda b: (0, 0)),
            pl.BlockSpec((1, 4, D), lambda b: (b, 0, 0)),
            pl.BlockSpec((1, C, D), lambda b: (b, 0, 0)),
            pl.BlockSpec((1, 4, D), lambda b: (b, 0, 0)),
            pl.BlockSpec((1, C, D), lambda b: (b, 0, 0)),
        ],
        out_specs=pl.BlockSpec((1, 1), lambda b: (0, 0)),
        out_shape=jax.ShapeDtypeStruct((1, 1), jnp.float32),
        scratch_shapes=[
            pltpu.VMEM((B, D), jnp.float32),
            pltpu.VMEM((B, D), jnp.float32),
        ],
    )(posf, xloc, xconf, gloc_t, gconf_t)
    return out[0, 0]


# speedup vs baseline: 5.1530x; 1.0002x over previous
import jax
import jax.numpy as jnp
from jax.experimental import pallas as pl
from jax.experimental.pallas import tpu as pltpu

B, D, C = 32, 8732, 21
ALPHA = 1.0
NEG_FACTOR = 3.0
N_ITERS = 16


def _ssd_step(pos_ref, xloc_ref, xconf_ref, gloc_ref, gconf_ref, out_ref,
              pn_s, l1_s):
    b = pl.program_id(0)

    xloc = xloc_ref[0]
    xconf = xconf_ref[0]
    gloc = gloc_ref[0]
    gconf = gconf_ref[0]

    d = xloc - gloc
    ad = jnp.abs(d)
    sl1 = jnp.where(ad < 1.0, 0.5 * d * d, ad - 0.5)
    l1 = jnp.sum(sl1, axis=0, keepdims=True)

    e = jnp.exp(xconf)
    lse = jnp.log(jnp.sum(e, axis=0, keepdims=True,
                          dtype=jnp.float32))
    gsum = jnp.sum(gconf, axis=0, keepdims=True,
                   dtype=jnp.float32)
    dot = jnp.sum(gconf * xconf, axis=0, keepdims=True,
                  dtype=jnp.float32)
    pn = gsum * lse - dot

    pn_s[pl.ds(b, 1), :] = pn
    l1_s[pl.ds(b, 1), :] = l1

    @pl.when(b == B - 1)
    def _final():
        pn_all = pn_s[:, :]
        l1a = l1_s[:, :]
        posa = pos_ref[:, :]
        is_pos = posa > 0.5

        n_pos = jnp.sum(posa, axis=1, keepdims=True)
        p_sum = jnp.sum(jnp.where(is_pos, pn_all, 0.0), axis=1,
                        keepdims=True)
        l1_sum = jnp.sum(jnp.where(is_pos, l1a, 0.0), axis=1,
                         keepdims=True)
        neg_cnt = jnp.float32(D) - n_pos
        k = jnp.minimum(neg_cnt, NEG_FACTOR * n_pos)

        negv = jnp.where(is_pos, -1.0, pn_all)

        lo0 = jnp.full((B, 1), -0.5, jnp.float32)
        hi0 = jnp.max(negv, axis=1, keepdims=True) + 1.0

        def body(_, carry):
            lo, hi = carry
            mid = 0.5 * (lo + hi)
            cnt = jnp.sum(jnp.where(negv > mid, 1.0, 0.0), axis=1,
                          keepdims=True)
            ge = cnt >= k
            return jnp.where(ge, mid, lo), jnp.where(ge, hi, mid)

        lo, hi = jax.lax.fori_loop(0, N_ITERS, body, (lo0, hi0))
        gt = negv > hi
        c = jnp.sum(jnp.where(gt, 1.0, 0.0), axis=1, keepdims=True)
        sum_gt = jnp.sum(jnp.where(gt, negv, 0.0), axis=1, keepdims=True)
        n_sum = sum_gt + (k - c) * hi
        n_sum = jnp.where(k > 0.0, n_sum, 0.0)

        safe_n = jnp.maximum(n_pos, 1.0)
        has_pos = n_pos > 0.0
        conf_loss = jnp.where(has_pos, (p_sum + n_sum) / safe_n, 0.0)
        loc_loss = jnp.where(has_pos, l1_sum / safe_n, 0.0)
        total = jnp.sum(conf_loss + ALPHA * loc_loss, axis=0,
                        keepdims=True) / jnp.float32(B)
        out_ref[:, :] = total


def kernel(pos_indicator, predicts, gt_loc, gt_conf):
    bf = jnp.bfloat16
    posf = pos_indicator.astype(jnp.float32)
    xloc = jnp.transpose(predicts[:, :, :4], (0, 2, 1))
    xconf = jnp.transpose(predicts[:, :, 4:], (0, 2, 1)).astype(bf)
    gloc_t = jnp.transpose(gt_loc, (0, 2, 1))
    gconf_t = jnp.transpose(gt_conf, (0, 2, 1)).astype(bf)

    out = pl.pallas_call(
        _ssd_step,
        grid=(B,),
        in_specs=[
            pl.BlockSpec((B, D), lambda b: (0, 0)),
            pl.BlockSpec((1, 4, D), lambda b: (b, 0, 0)),
            pl.BlockSpec((1, C, D), lambda b: (b, 0, 0)),
            pl.BlockSpec((1, 4, D), lambda b: (b, 0, 0)),
            pl.BlockSpec((1, C, D), lambda b: (b, 0, 0)),
        ],
        out_specs=pl.BlockSpec((1, 1), lambda b: (0, 0)),
        out_shape=jax.ShapeDtypeStruct((1, 1), jnp.float32),
        scratch_shapes=[
            pltpu.VMEM((B, D), jnp.float32),
            pltpu.VMEM((B, D), jnp.float32),
        ],
    )(posf, xloc, xconf, gloc_t, gconf_t)
    return out[0, 0]


# speedup vs baseline: 5.1643x; 1.0022x over previous
import jax
import jax.numpy as jnp
from jax.experimental import pallas as pl
from jax.experimental.pallas import tpu as pltpu

B, D, C = 32, 8732, 21
ALPHA = 1.0
NEG_FACTOR = 3.0
N_ITERS = 16


def _ssd_step(pos_ref, xloc_ref, xconf_ref, gloc_ref, gconf_ref, out_ref,
              pn_s):
    b = pl.program_id(0)

    xloc = xloc_ref[0]
    xconf = xconf_ref[0]
    gloc = gloc_ref[0]
    gconf = gconf_ref[0]

    d = xloc - gloc
    ad = jnp.abs(d)
    sl1 = jnp.where(ad < 1.0, 0.5 * d * d, ad - 0.5)
    l1 = jnp.sum(sl1, axis=0, keepdims=True)

    e = jnp.exp(xconf)
    lse = jnp.log(jnp.sum(e, axis=0, keepdims=True,
                          dtype=jnp.float32))
    gsum = jnp.sum(gconf, axis=0, keepdims=True,
                   dtype=jnp.float32)
    dot = jnp.sum(gconf * xconf, axis=0, keepdims=True,
                  dtype=jnp.float32)
    pn = gsum * lse - dot

    posrow = pos_ref[pl.ds(b, 1), :]
    pn_s[pl.ds(b, 1), :] = jnp.where(posrow > 0.5, pn + ALPHA * l1, pn)

    @pl.when(b == B - 1)
    def _final():
        t_all = pn_s[:, :]
        posa = pos_ref[:, :]
        is_pos = posa > 0.5

        n_pos = jnp.sum(posa, axis=1, keepdims=True)
        pl1_sum = jnp.sum(jnp.where(is_pos, t_all, 0.0), axis=1,
                          keepdims=True)
        neg_cnt = jnp.float32(D) - n_pos
        k = jnp.minimum(neg_cnt, NEG_FACTOR * n_pos)

        negv = jnp.where(is_pos, -1.0, t_all)

        lo0 = jnp.full((B, 1), -0.5, jnp.float32)
        hi0 = jnp.max(negv, axis=1, keepdims=True) + 1.0

        def body(_, carry):
            lo, hi = carry
            mid = 0.5 * (lo + hi)
            cnt = jnp.sum(jnp.where(negv > mid, 1.0, 0.0), axis=1,
                          keepdims=True)
            ge = cnt >= k
            return jnp.where(ge, mid, lo), jnp.where(ge, hi, mid)

        lo, hi = jax.lax.fori_loop(0, N_ITERS, body, (lo0, hi0))
        gt = negv > hi
        c = jnp.sum(jnp.where(gt, 1.0, 0.0), axis=1, keepdims=True)
        sum_gt = jnp.sum(jnp.where(gt, negv, 0.0), axis=1, keepdims=True)
        n_sum = sum_gt + (k - c) * hi
        n_sum = jnp.where(k > 0.0, n_sum, 0.0)

        safe_n = jnp.maximum(n_pos, 1.0)
        has_pos = n_pos > 0.0
        row_loss = jnp.where(has_pos, (pl1_sum + n_sum) / safe_n, 0.0)
        total = jnp.sum(row_loss, axis=0,
                        keepdims=True) / jnp.float32(B)
        out_ref[:, :] = total


def kernel(pos_indicator, predicts, gt_loc, gt_conf):
    bf = jnp.bfloat16
    posf = pos_indicator.astype(jnp.float32)
    xloc = jnp.transpose(predicts[:, :, :4], (0, 2, 1))
    xconf = jnp.transpose(predicts[:, :, 4:], (0, 2, 1)).astype(bf)
    gloc_t = jnp.transpose(gt_loc, (0, 2, 1))
    gconf_t = jnp.transpose(gt_conf, (0, 2, 1)).astype(bf)

    out = pl.pallas_call(
        _ssd_step,
        grid=(B,),
        in_specs=[
            pl.BlockSpec((B, D), lambda b: (0, 0)),
            pl.BlockSpec((1, 4, D), lambda b: (b, 0, 0)),
            pl.BlockSpec((1, C, D), lambda b: (b, 0, 0)),
            pl.BlockSpec((1, 4, D), lambda b: (b, 0, 0)),
            pl.BlockSpec((1, C, D), lambda b: (b, 0, 0)),
        ],
        out_specs=pl.BlockSpec((1, 1), lambda b: (0, 0)),
        out_shape=jax.ShapeDtypeStruct((1, 1), jnp.float32),
        scratch_shapes=[
            pltpu.VMEM((B, D), jnp.float32),
        ],
    )(posf, xloc, xconf, gloc_t, gconf_t)
    return out[0, 0]
